# baseline (speedup 1.0000x reference)
import functools

import jax
import jax.numpy as jnp
from jax import lax
from jax.experimental import pallas as pl
from jax.experimental.pallas import tpu as pltpu
from jax.experimental.pallas import tpu_sc as plsc

T = 2048
D = 1024
F = 2048
E = 8
B = 128
G = T // B + E
GB = G * B

NC, NS = 2, 16
NW = NC * NS
RPW = GB // NW
OUT_PAD = 8
VW = 128


def _router_kernel(x_ref, wr_ref, idx_ref, val_ref, aux_ref):
    x = x_ref[...]
    wr = wr_ref[...]
    logits = jnp.dot(x, wr, preferred_element_type=jnp.float32)
    m = jnp.max(logits, axis=-1, keepdims=True)
    ex = jnp.exp(logits - m)
    s = jnp.sum(ex, axis=-1, keepdims=True)
    gates = ex / s
    iota = jax.lax.broadcasted_iota(jnp.int32, logits.shape, 1)
    idx = jnp.min(jnp.where(logits >= m, iota, E), axis=-1)
    one_hot = (iota == idx[:, None]).astype(jnp.float32)
    importance = jnp.sum(gates, axis=0)
    load = jnp.sum(one_hot, axis=0)
    aux = (E / (T * T)) * jnp.sum(importance * load)
    idx_ref[...] = idx[:, None]
    val_ref[...] = jnp.broadcast_to(1.0 / s, (T, VW))
    aux_ref[...] = jnp.reshape(aux, (1, 1))


@functools.lru_cache(maxsize=1)
def _sc_kernels():
    mesh = plsc.VectorSubcoreMesh(core_axis_name="c", subcore_axis_name="s",
                                  num_cores=NC, num_subcores=NS)

    @functools.partial(
        pl.kernel,
        out_type=(jax.ShapeDtypeStruct((GB, D), jnp.float32),
                  jax.ShapeDtypeStruct((GB, VW), jnp.float32)),
        mesh=mesh,
        scratch_types=[
            pltpu.VMEM((RPW,), jnp.int32),
            pltpu.VMEM((RPW, D), jnp.float32),
            pltpu.VMEM((RPW, VW), jnp.float32),
            pltpu.SemaphoreType.DMA,
            pltpu.SemaphoreType.DMA,
        ],
    )
    def sc_gather(x_hbm, val_hbm, gidx_hbm, xs_hbm, vals_hbm,
                  idx_v, rows_v, vrows_v, sem1, sem2):
        wid = lax.axis_index("s") * NC + lax.axis_index("c")
        base = wid * RPW
        pltpu.sync_copy(gidx_hbm.at[pl.ds(base, RPW)], idx_v)
        cp1 = pltpu.async_copy(x_hbm.at[idx_v], rows_v, sem1)
        cp2 = pltpu.async_copy(val_hbm.at[idx_v], vrows_v, sem2)
        cp1.wait()
        cp2.wait()
        pltpu.sync_copy(rows_v, xs_hbm.at[pl.ds(base, RPW)])
        pltpu.sync_copy(vrows_v, vals_hbm.at[pl.ds(base, RPW)])

    @functools.partial(
        pl.kernel,
        out_type=jax.ShapeDtypeStruct((T + OUT_PAD, D), jnp.float32),
        mesh=mesh,
        scratch_types=[
            pltpu.VMEM((RPW,), jnp.int32),
            pltpu.VMEM((RPW, D), jnp.float32),
            pltpu.SemaphoreType.DMA,
        ],
    )
    def sc_scatter(ys_hbm, sidx_hbm, out_hbm, idx_v, rows_v, sem):
        wid = lax.axis_index("s") * NC + lax.axis_index("c")
        base = wid * RPW
        pltpu.sync_copy(sidx_hbm.at[pl.ds(base, RPW)], idx_v)
        pltpu.sync_copy(ys_hbm.at[pl.ds(base, RPW)], rows_v)
        pltpu.async_copy(rows_v, out_hbm.at[idx_v], sem).wait()

    return sc_gather, sc_scatter


def _ffn_kernel(be_ref, bvalid_ref,
                xs_ref, w1_ref, w2_ref, vals_ref,
                ys_ref, h_ref):
    g = pl.program_id(0)

    @pl.when(bvalid_ref[g] > 0)
    def _():
        h_ref[...] = jnp.maximum(
            jnp.dot(xs_ref[...], w1_ref[0], preferred_element_type=jnp.float32),
            0.0)
        vals = vals_ref[...]
        ys_ref[...] = jnp.dot(h_ref[...], w2_ref[0],
                              preferred_element_type=jnp.float32) * vals[:, :1]


@jax.jit
def kernel(x, Wr, W1, W2):
    idx2, val16, aux2 = pl.pallas_call(
        _router_kernel,
        out_shape=(
            jax.ShapeDtypeStruct((T, 1), jnp.int32),
            jax.ShapeDtypeStruct((T, VW), jnp.float32),
            jax.ShapeDtypeStruct((1, 1), jnp.float32),
        ),
    )(x, Wr)
    top_idx = idx2[:, 0]

    perm = jnp.argsort(top_idx, stable=True).astype(jnp.int32)
    counts = jnp.sum((top_idx[:, None] == jnp.arange(E)[None, :]).astype(jnp.int32),
                     axis=0)
    offsets = jnp.concatenate([jnp.zeros((1,), jnp.int32),
                               jnp.cumsum(counts)[:-1].astype(jnp.int32)])
    nblk = (counts + B - 1) // B
    blk_cum = jnp.concatenate([jnp.zeros((1,), jnp.int32),
                               jnp.cumsum(nblk)[:-1].astype(jnp.int32)])
    gid = jnp.arange(G, dtype=jnp.int32)
    be = jnp.sum((blk_cum[None, :] <= gid[:, None]).astype(jnp.int32), axis=1) - 1
    k = gid - blk_cum[be]
    bstart = offsets[be] + k * B
    bvalid = jnp.clip(counts[be] - k * B, 0, B)

    j = jnp.arange(GB, dtype=jnp.int32)
    gj = j // B
    ij = j % B
    src = bstart[gj] + ij
    valid_j = ij < bvalid[gj]
    gidx = perm[jnp.clip(src, 0, T - 1)]
    sidx = jnp.where(valid_j, gidx, T + (ij & (OUT_PAD - 1)))

    sc_gather, sc_scatter = _sc_kernels()
    xs, vals = sc_gather(x, val16, gidx)

    grid_spec = pltpu.PrefetchScalarGridSpec(
        num_scalar_prefetch=2,
        grid=(G,),
        in_specs=[
            pl.BlockSpec((B, D), lambda g, be, bv: (g, 0)),
            pl.BlockSpec((1, D, F), lambda g, be, bv: (be[g], 0, 0)),
            pl.BlockSpec((1, F, D), lambda g, be, bv: (be[g], 0, 0)),
            pl.BlockSpec((B, VW), lambda g, be, bv: (g, 0)),
        ],
        out_specs=pl.BlockSpec((B, D), lambda g, be, bv: (g, 0)),
        scratch_shapes=[pltpu.VMEM((B, F), jnp.float32)],
    )
    ys = pl.pallas_call(
        _ffn_kernel,
        grid_spec=grid_spec,
        out_shape=jax.ShapeDtypeStruct((GB, D), jnp.float32),
        compiler_params=pltpu.CompilerParams(
            dimension_semantics=("arbitrary",)),
    )(be, bvalid, xs, W1, W2, vals)

    out_pad = sc_scatter(ys, sidx)
    return out_pad[:T], top_idx, aux2[0, 0]


# speedup vs baseline: 1.7184x; 1.7184x over previous
import functools

import jax
import jax.numpy as jnp
from jax.experimental import pallas as pl
from jax.experimental.pallas import tpu as pltpu

T = 2048
D = 1024
F = 2048
E = 8
B = 128
G = T // B + E


def _router_kernel(x_ref, wr_ref, idx_ref, val_ref, aux_ref):
    x = x_ref[...]
    wr = wr_ref[...]
    logits = jnp.dot(x, wr, preferred_element_type=jnp.float32)
    m = jnp.max(logits, axis=-1, keepdims=True)
    ex = jnp.exp(logits - m)
    s = jnp.sum(ex, axis=-1, keepdims=True)
    gates = ex / s
    iota = jax.lax.broadcasted_iota(jnp.int32, logits.shape, 1)
    idx = jnp.min(jnp.where(logits >= m, iota, E), axis=-1)
    one_hot = (iota == idx[:, None]).astype(jnp.float32)
    importance = jnp.sum(gates, axis=0)
    load = jnp.sum(one_hot, axis=0)
    aux = (E / (T * T)) * jnp.sum(importance * load)
    idx_ref[...] = idx[:, None]
    val_ref[...] = 1.0 / s
    aux_ref[...] = jnp.reshape(aux, (1, 1))


def _ffn_kernel(perm_ref, be_ref, bstart_ref, bvalid_ref,
                rfirst_ref, fetch_ref, ne_ref, slot_ref,
                x_ref, w1_any, w2_any, val_ref,
                out_ref,
                w1b_ref, w2b_ref, sem1, sem2,
                xb_ref, vb_ref, h_ref, yb_ref):
    g = pl.program_id(0)
    start = bstart_ref[g]
    valid = bvalid_ref[g]
    slot = slot_ref[g]

    @pl.when(g == 0)
    def _():
        e0 = be_ref[0]
        pltpu.make_async_copy(w1_any.at[e0], w1b_ref.at[0], sem1.at[0]).start()
        pltpu.make_async_copy(w2_any.at[e0], w2b_ref.at[0], sem2.at[0]).start()

    @pl.when(fetch_ref[g] == 1)
    def _():
        ne = ne_ref[g]
        ns = 1 - slot
        pltpu.make_async_copy(w1_any.at[ne], w1b_ref.at[ns], sem1.at[ns]).start()
        pltpu.make_async_copy(w2_any.at[ne], w2b_ref.at[ns], sem2.at[ns]).start()

    @pl.when(valid > 0)
    def _():
        def gather(i, _):
            r = jnp.minimum(start + i, T - 1)
            tok = perm_ref[r]
            xb_ref[i, :] = x_ref[tok, :]
            vb_ref[i, :] = val_ref[tok, :]
            return 0
        jax.lax.fori_loop(0, B, gather, 0, unroll=8)

    @pl.when(rfirst_ref[g] == 1)
    def _():
        e = be_ref[g]
        pltpu.make_async_copy(w1_any.at[e], w1b_ref.at[slot], sem1.at[slot]).wait()

    @pl.when(valid > 0)
    def _():
        h_ref[...] = jnp.maximum(
            jnp.dot(xb_ref[...], w1b_ref[slot],
                    preferred_element_type=jnp.float32),
            0.0)

    @pl.when(rfirst_ref[g] == 1)
    def _():
        e = be_ref[g]
        pltpu.make_async_copy(w2_any.at[e], w2b_ref.at[slot], sem2.at[slot]).wait()

    @pl.when(valid > 0)
    def _():
        yb_ref[...] = jnp.dot(h_ref[...], w2b_ref[slot],
                              preferred_element_type=jnp.float32) * vb_ref[...]

        def scatter(i, _):
            @pl.when(i < valid)
            def _():
                tok = perm_ref[start + i]
                out_ref[tok, :] = yb_ref[i, :]
            return 0
        jax.lax.fori_loop(0, B, scatter, 0, unroll=8)


@jax.jit
def kernel(x, Wr, W1, W2):
    idx2, val2, aux2 = pl.pallas_call(
        _router_kernel,
        out_shape=(
            jax.ShapeDtypeStruct((T, 1), jnp.int32),
            jax.ShapeDtypeStruct((T, 1), jnp.float32),
            jax.ShapeDtypeStruct((1, 1), jnp.float32),
        ),
    )(x, Wr)
    top_idx = idx2[:, 0]

    perm = jnp.argsort(top_idx, stable=True).astype(jnp.int32)
    counts = jnp.sum((top_idx[:, None] == jnp.arange(E)[None, :]).astype(jnp.int32),
                     axis=0)
    offsets = jnp.concatenate([jnp.zeros((1,), jnp.int32),
                               jnp.cumsum(counts)[:-1].astype(jnp.int32)])
    nblk = (counts + B - 1) // B
    blk_cum = jnp.concatenate([jnp.zeros((1,), jnp.int32),
                               jnp.cumsum(nblk)[:-1].astype(jnp.int32)])
    gid = jnp.arange(G, dtype=jnp.int32)
    be = jnp.sum((blk_cum[None, :] <= gid[:, None]).astype(jnp.int32), axis=1) - 1
    k = gid - blk_cum[be]
    bstart = offsets[be] + k * B
    bvalid = jnp.clip(counts[be] - k * B, 0, B)

    rfirst = jnp.concatenate([jnp.ones((1,), jnp.int32),
                              (be[1:] != be[:-1]).astype(jnp.int32)])
    run_idx = jnp.cumsum(rfirst) - 1
    slot = (run_idx % 2).astype(jnp.int32)
    later = gid[None, :] > gid[:, None]
    differs = be[None, :] != be[:, None]
    nxt_change = jnp.min(jnp.where(later & differs, gid[None, :], G - 1),
                         axis=1)
    ne = be[nxt_change]
    fetch = rfirst * (ne != be).astype(jnp.int32)

    grid_spec = pltpu.PrefetchScalarGridSpec(
        num_scalar_prefetch=8,
        grid=(G,),
        in_specs=[
            pl.BlockSpec((T, D), lambda g, *_: (0, 0)),
            pl.BlockSpec(memory_space=pltpu.MemorySpace.HBM),
            pl.BlockSpec(memory_space=pltpu.MemorySpace.HBM),
            pl.BlockSpec((T, 1), lambda g, *_: (0, 0)),
        ],
        out_specs=pl.BlockSpec((T, D), lambda g, *_: (0, 0)),
        scratch_shapes=[
            pltpu.VMEM((2, D, F), jnp.float32),
            pltpu.VMEM((2, F, D), jnp.float32),
            pltpu.SemaphoreType.DMA((2,)),
            pltpu.SemaphoreType.DMA((2,)),
            pltpu.VMEM((B, D), jnp.float32),
            pltpu.VMEM((B, 1), jnp.float32),
            pltpu.VMEM((B, F), jnp.float32),
            pltpu.VMEM((B, D), jnp.float32),
        ],
    )
    out = pl.pallas_call(
        _ffn_kernel,
        grid_spec=grid_spec,
        out_shape=jax.ShapeDtypeStruct((T, D), jnp.float32),
        compiler_params=pltpu.CompilerParams(
            dimension_semantics=("arbitrary",)),
    )(perm, be, bstart, bvalid, rfirst, fetch, ne, slot, x, W1, W2, val2)

    return out, top_idx, aux2[0, 0]


# speedup vs baseline: 2.2867x; 1.3307x over previous
import functools

import jax
import jax.numpy as jnp
from jax.experimental import pallas as pl
from jax.experimental.pallas import tpu as pltpu

T = 2048
D = 1024
F = 2048
E = 8
B = 128
G = T // B + E


def _router_kernel(x_ref, wr_ref, idx_ref, val_ref, aux_ref):
    x = x_ref[...]
    wr = wr_ref[...]
    logits = jnp.dot(x, wr, preferred_element_type=jnp.float32)
    m = jnp.max(logits, axis=-1, keepdims=True)
    ex = jnp.exp(logits - m)
    s = jnp.sum(ex, axis=-1, keepdims=True)
    gates = ex / s
    iota = jax.lax.broadcasted_iota(jnp.int32, logits.shape, 1)
    idx = jnp.min(jnp.where(logits >= m, iota, E), axis=-1)
    one_hot = (iota == idx[:, None]).astype(jnp.float32)
    importance = jnp.sum(gates, axis=0)
    load = jnp.sum(one_hot, axis=0)
    aux = (E / (T * T)) * jnp.sum(importance * load)
    idx_ref[...] = idx[:, None]
    val_ref[...] = 1.0 / s
    aux_ref[...] = jnp.reshape(aux, (1, 1))


def _ffn_kernel(perm_ref, be_ref, bstart_ref, bvalid_ref,
                rfirst_ref, fetch_ref, ne_ref, slot_ref,
                x_ref, w1_any, w2_any, val_ref,
                out_ref,
                w1b_ref, w2b_ref, sem1, sem2,
                xb_ref, vb_ref, h_ref, yb_ref):
    g = pl.program_id(0)
    start = bstart_ref[g]
    valid = bvalid_ref[g]
    slot = slot_ref[g]

    @pl.when(g == 0)
    def _():
        e0 = be_ref[0]
        pltpu.make_async_copy(w1_any.at[e0], w1b_ref.at[0], sem1.at[0]).start()
        pltpu.make_async_copy(w2_any.at[e0], w2b_ref.at[0], sem2.at[0]).start()

    @pl.when(fetch_ref[g] == 1)
    def _():
        ne = ne_ref[g]
        ns = 1 - slot
        pltpu.make_async_copy(w1_any.at[ne], w1b_ref.at[ns], sem1.at[ns]).start()
        pltpu.make_async_copy(w2_any.at[ne], w2b_ref.at[ns], sem2.at[ns]).start()

    @pl.when(valid > 0)
    def _():
        nfull = (valid // 8) * 8

        def gather8(c, _):
            b0 = c * 8
            for u in range(8):
                i = b0 + u
                tok = perm_ref[start + i]
                xb_ref[i, :] = x_ref[tok, :]
                vb_ref[i, :] = val_ref[tok, :]
            return 0
        jax.lax.fori_loop(0, valid // 8, gather8, 0)

        def gather1(i, _):
            tok = perm_ref[start + i]
            xb_ref[i, :] = x_ref[tok, :]
            vb_ref[i, :] = val_ref[tok, :]
            return 0
        jax.lax.fori_loop(nfull, valid, gather1, 0)

    @pl.when(rfirst_ref[g] == 1)
    def _():
        e = be_ref[g]
        pltpu.make_async_copy(w1_any.at[e], w1b_ref.at[slot], sem1.at[slot]).wait()

    @pl.when(valid > 0)
    def _():
        h_ref[...] = jnp.maximum(
            jnp.dot(xb_ref[...], w1b_ref[slot],
                    preferred_element_type=jnp.float32),
            0.0)

    @pl.when(rfirst_ref[g] == 1)
    def _():
        e = be_ref[g]
        pltpu.make_async_copy(w2_any.at[e], w2b_ref.at[slot], sem2.at[slot]).wait()

    @pl.when(valid > 0)
    def _():
        yb_ref[...] = jnp.dot(h_ref[...], w2b_ref[slot],
                              preferred_element_type=jnp.float32) * vb_ref[...]

        nfull = (valid // 8) * 8

        def scatter8(c, _):
            b0 = c * 8
            for u in range(8):
                i = b0 + u
                tok = perm_ref[start + i]
                out_ref[tok, :] = yb_ref[i, :]
            return 0
        jax.lax.fori_loop(0, valid // 8, scatter8, 0)

        def scatter1(i, _):
            tok = perm_ref[start + i]
            out_ref[tok, :] = yb_ref[i, :]
            return 0
        jax.lax.fori_loop(nfull, valid, scatter1, 0)


@jax.jit
def kernel(x, Wr, W1, W2):
    idx2, val2, aux2 = pl.pallas_call(
        _router_kernel,
        out_shape=(
            jax.ShapeDtypeStruct((T, 1), jnp.int32),
            jax.ShapeDtypeStruct((T, 1), jnp.float32),
            jax.ShapeDtypeStruct((1, 1), jnp.float32),
        ),
    )(x, Wr)
    top_idx = idx2[:, 0]

    perm = jnp.argsort(top_idx, stable=True).astype(jnp.int32)
    counts = jnp.sum((top_idx[:, None] == jnp.arange(E)[None, :]).astype(jnp.int32),
                     axis=0)
    offsets = jnp.concatenate([jnp.zeros((1,), jnp.int32),
                               jnp.cumsum(counts)[:-1].astype(jnp.int32)])
    nblk = (counts + B - 1) // B
    blk_cum = jnp.concatenate([jnp.zeros((1,), jnp.int32),
                               jnp.cumsum(nblk)[:-1].astype(jnp.int32)])
    gid = jnp.arange(G, dtype=jnp.int32)
    be = jnp.sum((blk_cum[None, :] <= gid[:, None]).astype(jnp.int32), axis=1) - 1
    k = gid - blk_cum[be]
    bstart = offsets[be] + k * B
    bvalid = jnp.clip(counts[be] - k * B, 0, B)

    rfirst = jnp.concatenate([jnp.ones((1,), jnp.int32),
                              (be[1:] != be[:-1]).astype(jnp.int32)])
    run_idx = jnp.cumsum(rfirst) - 1
    slot = (run_idx % 2).astype(jnp.int32)
    later = gid[None, :] > gid[:, None]
    differs = be[None, :] != be[:, None]
    nxt_change = jnp.min(jnp.where(later & differs, gid[None, :], G - 1),
                         axis=1)
    ne = be[nxt_change]
    fetch = rfirst * (ne != be).astype(jnp.int32)

    grid_spec = pltpu.PrefetchScalarGridSpec(
        num_scalar_prefetch=8,
        grid=(G,),
        in_specs=[
            pl.BlockSpec((T, D), lambda g, *_: (0, 0)),
            pl.BlockSpec(memory_space=pltpu.MemorySpace.HBM),
            pl.BlockSpec(memory_space=pltpu.MemorySpace.HBM),
            pl.BlockSpec((T, 1), lambda g, *_: (0, 0)),
        ],
        out_specs=pl.BlockSpec((T, D), lambda g, *_: (0, 0)),
        scratch_shapes=[
            pltpu.VMEM((2, D, F), jnp.float32),
            pltpu.VMEM((2, F, D), jnp.float32),
            pltpu.SemaphoreType.DMA((2,)),
            pltpu.SemaphoreType.DMA((2,)),
            pltpu.VMEM((B, D), jnp.float32),
            pltpu.VMEM((B, 1), jnp.float32),
            pltpu.VMEM((B, F), jnp.float32),
            pltpu.VMEM((B, D), jnp.float32),
        ],
    )
    out = pl.pallas_call(
        _ffn_kernel,
        grid_spec=grid_spec,
        out_shape=jax.ShapeDtypeStruct((T, D), jnp.float32),
        compiler_params=pltpu.CompilerParams(
            dimension_semantics=("arbitrary",)),
    )(perm, be, bstart, bvalid, rfirst, fetch, ne, slot, x, W1, W2, val2)

    return out, top_idx, aux2[0, 0]


# speedup vs baseline: 2.4184x; 1.0576x over previous
import functools

import jax
import jax.numpy as jnp
from jax.experimental import pallas as pl
from jax.experimental.pallas import tpu as pltpu

T = 2048
D = 1024
F = 2048
E = 8
B = 128
G = T // B + E


def _router_kernel(x_ref, wr_ref, idx_ref, val_ref, aux_ref):
    x = x_ref[...]
    wr = wr_ref[...]
    logits = jnp.dot(x, wr, preferred_element_type=jnp.float32)
    m = jnp.max(logits, axis=-1, keepdims=True)
    ex = jnp.exp(logits - m)
    s = jnp.sum(ex, axis=-1, keepdims=True)
    gates = ex / s
    iota = jax.lax.broadcasted_iota(jnp.int32, logits.shape, 1)
    idx = jnp.min(jnp.where(logits >= m, iota, E), axis=-1)
    one_hot = (iota == idx[:, None]).astype(jnp.float32)
    importance = jnp.sum(gates, axis=0)
    load = jnp.sum(one_hot, axis=0)
    aux = (E / (T * T)) * jnp.sum(importance * load)
    idx_ref[...] = idx[:, None]
    val_ref[...] = 1.0 / s
    aux_ref[...] = jnp.reshape(aux, (1, 1))


def _ffn_kernel(perm_ref, be_ref, bstart_ref, bvalid_ref,
                rfirst_ref, fetch_ref, ne_ref, slot_ref,
                x_ref, w1_any, w2_any, val_ref,
                out_ref,
                w1b_ref, w2b_ref, sem1, sem2,
                xb_ref, vb_ref, h_ref, yb_ref):
    g = pl.program_id(0)
    start = bstart_ref[g]
    valid = bvalid_ref[g]
    slot = slot_ref[g]

    @pl.when(g == 0)
    def _():
        e0 = be_ref[0]
        pltpu.make_async_copy(w1_any.at[e0], w1b_ref.at[0], sem1.at[0]).start()
        pltpu.make_async_copy(w2_any.at[e0], w2b_ref.at[0], sem2.at[0]).start()

    @pl.when(fetch_ref[g] == 1)
    def _():
        ne = ne_ref[g]
        ns = 1 - slot
        pltpu.make_async_copy(w1_any.at[ne], w1b_ref.at[ns], sem1.at[ns]).start()
        pltpu.make_async_copy(w2_any.at[ne], w2b_ref.at[ns], sem2.at[ns]).start()

    @pl.when(valid > 0)
    def _():
        nfull = (valid // 8) * 8

        def gather8(c, _):
            b0 = c * 8
            for u in range(8):
                i = b0 + u
                tok = perm_ref[start + i]
                xb_ref[i, :] = x_ref[tok, :]
                vb_ref[i, :] = val_ref[tok, :]
            return 0
        jax.lax.fori_loop(0, valid // 8, gather8, 0)

        def gather1(i, _):
            tok = perm_ref[start + i]
            xb_ref[i, :] = x_ref[tok, :]
            vb_ref[i, :] = val_ref[tok, :]
            return 0
        jax.lax.fori_loop(nfull, valid, gather1, 0)

    @pl.when(rfirst_ref[g] == 1)
    def _():
        e = be_ref[g]
        pltpu.make_async_copy(w1_any.at[e], w1b_ref.at[slot], sem1.at[slot]).wait()
        pltpu.make_async_copy(w2_any.at[e], w2b_ref.at[slot], sem2.at[slot]).wait()

    @pl.when(valid > 0)
    def _():
        h_ref[...] = jnp.maximum(
            jnp.dot(xb_ref[...], w1b_ref[slot],
                    preferred_element_type=jnp.float32),
            0.0)
        yb_ref[...] = jnp.dot(h_ref[...], w2b_ref[slot],
                              preferred_element_type=jnp.float32) * vb_ref[...]

        nfull = (valid // 8) * 8

        def scatter8(c, _):
            b0 = c * 8
            for u in range(8):
                i = b0 + u
                tok = perm_ref[start + i]
                out_ref[tok, :] = yb_ref[i, :]
            return 0
        jax.lax.fori_loop(0, valid // 8, scatter8, 0)

        def scatter1(i, _):
            tok = perm_ref[start + i]
            out_ref[tok, :] = yb_ref[i, :]
            return 0
        jax.lax.fori_loop(nfull, valid, scatter1, 0)


@jax.jit
def kernel(x, Wr, W1, W2):
    idx2, val2, aux2 = pl.pallas_call(
        _router_kernel,
        out_shape=(
            jax.ShapeDtypeStruct((T, 1), jnp.int32),
            jax.ShapeDtypeStruct((T, 1), jnp.float32),
            jax.ShapeDtypeStruct((1, 1), jnp.float32),
        ),
    )(x, Wr)
    top_idx = idx2[:, 0]

    perm = jnp.argsort(top_idx, stable=True).astype(jnp.int32)
    counts = jnp.sum((top_idx[:, None] == jnp.arange(E)[None, :]).astype(jnp.int32),
                     axis=0)
    offsets = jnp.concatenate([jnp.zeros((1,), jnp.int32),
                               jnp.cumsum(counts)[:-1].astype(jnp.int32)])
    nblk = (counts + B - 1) // B
    blk_cum = jnp.concatenate([jnp.zeros((1,), jnp.int32),
                               jnp.cumsum(nblk)[:-1].astype(jnp.int32)])
    gid = jnp.arange(G, dtype=jnp.int32)
    be = jnp.sum((blk_cum[None, :] <= gid[:, None]).astype(jnp.int32), axis=1) - 1
    k = gid - blk_cum[be]
    bstart = offsets[be] + k * B
    bvalid = jnp.clip(counts[be] - k * B, 0, B)

    rfirst = jnp.concatenate([jnp.ones((1,), jnp.int32),
                              (be[1:] != be[:-1]).astype(jnp.int32)])
    run_idx = jnp.cumsum(rfirst) - 1
    slot = (run_idx % 2).astype(jnp.int32)
    later = gid[None, :] > gid[:, None]
    differs = be[None, :] != be[:, None]
    nxt_change = jnp.min(jnp.where(later & differs, gid[None, :], G - 1),
                         axis=1)
    ne = be[nxt_change]
    fetch = rfirst * (ne != be).astype(jnp.int32)

    grid_spec = pltpu.PrefetchScalarGridSpec(
        num_scalar_prefetch=8,
        grid=(G,),
        in_specs=[
            pl.BlockSpec((T, D), lambda g, *_: (0, 0)),
            pl.BlockSpec(memory_space=pltpu.MemorySpace.HBM),
            pl.BlockSpec(memory_space=pltpu.MemorySpace.HBM),
            pl.BlockSpec((T, 1), lambda g, *_: (0, 0)),
        ],
        out_specs=pl.BlockSpec((T, D), lambda g, *_: (0, 0)),
        scratch_shapes=[
            pltpu.VMEM((2, D, F), jnp.float32),
            pltpu.VMEM((2, F, D), jnp.float32),
            pltpu.SemaphoreType.DMA((2,)),
            pltpu.SemaphoreType.DMA((2,)),
            pltpu.VMEM((B, D), jnp.float32),
            pltpu.VMEM((B, 1), jnp.float32),
            pltpu.VMEM((B, F), jnp.float32),
            pltpu.VMEM((B, D), jnp.float32),
        ],
    )
    out = pl.pallas_call(
        _ffn_kernel,
        grid_spec=grid_spec,
        out_shape=jax.ShapeDtypeStruct((T, D), jnp.float32),
        compiler_params=pltpu.CompilerParams(
            dimension_semantics=("arbitrary",)),
    )(perm, be, bstart, bvalid, rfirst, fetch, ne, slot, x, W1, W2, val2)

    return out, top_idx, aux2[0, 0]
